# Initial kernel scaffold; baseline (speedup 1.0000x reference)
import jax
import jax.numpy as jnp
from jax.experimental import pallas as pl


def kernel(x, edge_index, W1, b1, gamma1, beta1, W2, b2):
    raise NotImplementedError("write your pallas kernel here")


# speedup vs baseline: 7.9948x; 7.9948x over previous
import functools

import jax
import jax.numpy as jnp
from jax import lax
from jax.experimental import pallas as pl
from jax.experimental.pallas import tpu as pltpu
from jax.experimental.pallas import tpu_sc as plsc

_N = 10000
_NP = 10240
_E = 160000
_EP = 163840
_B = 128
_RB = 1000
_NBLK = _N // _RB


@functools.lru_cache(maxsize=None)
def _build():
    mesh = plsc.VectorSubcoreMesh(core_axis_name="c", subcore_axis_name="s",
                                  num_cores=2, num_subcores=16)
    f32 = jnp.float32

    @functools.partial(
        pl.kernel, mesh=mesh,
        out_type=jax.ShapeDtypeStruct((2, _NP, 128), f32),
        scratch_types=[
            pltpu.VMEM((40, _B), jnp.int32),
            pltpu.VMEM((_B, 128), f32),
            pltpu.VMEM_SHARED((_NP, 128), f32),
        ],
    )
    def sc_deg(dst_hbm, ones_hbm, zero_hbm, out_hbm, dst_v, ones_v, acc_sh):
        c = lax.axis_index("c")
        s = lax.axis_index("s")
        w = c * 16 + s
        pltpu.sync_copy(zero_hbm.at[pl.ds(s * 640, 640)],
                        acc_sh.at[pl.ds(s * 640, 640)])
        pltpu.sync_copy(ones_hbm, ones_v)
        pltpu.sync_copy(dst_hbm.at[w], dst_v)
        plsc.subcore_barrier()

        def body(i, _):
            pltpu.sync_copy(ones_v, acc_sh.at[dst_v.at[i]], add=True)
            return 0

        lax.fori_loop(0, 40, body, 0)
        plsc.subcore_barrier()
        pltpu.sync_copy(acc_sh.at[pl.ds(s * 640, 640)],
                        out_hbm.at[c, pl.ds(s * 640, 640)])

    @functools.partial(
        pl.kernel, mesh=mesh,
        out_type=jax.ShapeDtypeStruct((2, _NP, 128), f32),
        scratch_types=[
            pltpu.VMEM((80, _B), jnp.int32),
            pltpu.VMEM((80, _B), jnp.int32),
            pltpu.VMEM((_B, 128), f32),
            pltpu.VMEM_SHARED((_NP, 128), f32),
        ],
    )
    def sc_agg1(h_hbm, src_hbm, dst_hbm, zero_hbm, out_hbm,
                src_v, dst_v, rows_v, acc_sh):
        c = lax.axis_index("c")
        s = lax.axis_index("s")
        pltpu.sync_copy(zero_hbm.at[pl.ds(s * 640, 640)],
                        acc_sh.at[pl.ds(s * 640, 640)])
        pltpu.sync_copy(src_hbm.at[c, s], src_v)
        pltpu.sync_copy(dst_hbm.at[s], dst_v)
        plsc.subcore_barrier()

        def body(i, _):
            pltpu.sync_copy(h_hbm.at[src_v.at[i]], rows_v)
            pltpu.sync_copy(rows_v, acc_sh.at[dst_v.at[i]], add=True)
            return 0

        lax.fori_loop(0, 80, body, 0)
        plsc.subcore_barrier()
        pltpu.sync_copy(acc_sh.at[pl.ds(s * 640, 640)],
                        out_hbm.at[c, pl.ds(s * 640, 640)])

    @functools.partial(
        pl.kernel, mesh=mesh,
        out_type=jax.ShapeDtypeStruct((2, _NP, 128), f32),
        scratch_types=[
            pltpu.VMEM((40, _B), jnp.int32),
            pltpu.VMEM((40, _B), jnp.int32),
            pltpu.VMEM((_B, 128), f32),
            pltpu.VMEM_SHARED((_NP, 128), f32),
        ],
    )
    def sc_agg2(g_hbm, src_hbm, dst_hbm, zero_hbm, out_hbm,
                src_v, dst_v, rows_v, acc_sh):
        c = lax.axis_index("c")
        s = lax.axis_index("s")
        w = c * 16 + s
        pltpu.sync_copy(zero_hbm.at[pl.ds(s * 640, 640)],
                        acc_sh.at[pl.ds(s * 640, 640)])
        pltpu.sync_copy(src_hbm.at[w], src_v)
        pltpu.sync_copy(dst_hbm.at[w], dst_v)
        plsc.subcore_barrier()

        def body(i, _):
            pltpu.sync_copy(g_hbm.at[src_v.at[i]], rows_v)
            pltpu.sync_copy(rows_v, acc_sh.at[dst_v.at[i]], add=True)
            return 0

        lax.fori_loop(0, 40, body, 0)
        plsc.subcore_barrier()
        pltpu.sync_copy(acc_sh.at[pl.ds(s * 640, 640)],
                        out_hbm.at[c, pl.ds(s * 640, 640)])

    def k1_body(x_ref, w1_ref, deg_ref, out_ref):
        h = jnp.dot(x_ref[...], w1_ref[...], preferred_element_type=f32)
        dsum = deg_ref[0, :, 0] + deg_ref[1, :, 0] + 1.0
        dinv = lax.rsqrt(dsum)[:, None]
        out_ref[0] = h[:, :128] * dinv
        out_ref[1] = h[:, 128:] * dinv

    k1 = pl.pallas_call(
        k1_body,
        grid=(_NBLK,),
        in_specs=[
            pl.BlockSpec((_RB, 256), lambda r: (r, 0)),
            pl.BlockSpec((256, 256), lambda r: (0, 0)),
            pl.BlockSpec((2, _RB, 128), lambda r: (0, r, 0)),
        ],
        out_specs=pl.BlockSpec((2, _RB, 128), lambda r: (0, r, 0)),
        out_shape=jax.ShapeDtypeStruct((2, _N, 128), f32),
    )

    def k2_body(a1_ref, h_ref, deg_ref, b1_ref, z_ref, sum_ref, sq_ref):
        dsum = deg_ref[0, :, 0] + deg_ref[1, :, 0] + 1.0
        dinv = lax.rsqrt(dsum)[:, None]
        z0 = (a1_ref[0] + h_ref[0]) * dinv + b1_ref[0]
        z1 = (a1_ref[1] + h_ref[1]) * dinv + b1_ref[1]
        z_ref[0] = z0
        z_ref[1] = z1
        sum_ref[...] = jnp.stack([jnp.sum(z0, 0), jnp.sum(z1, 0)])[None]
        sq_ref[...] = jnp.stack([jnp.sum(z0 * z0, 0),
                                 jnp.sum(z1 * z1, 0)])[None]

    k2 = pl.pallas_call(
        k2_body,
        grid=(_NBLK,),
        in_specs=[
            pl.BlockSpec((2, _RB, 128), lambda r: (0, r, 0)),
            pl.BlockSpec((2, _RB, 128), lambda r: (0, r, 0)),
            pl.BlockSpec((2, _RB, 128), lambda r: (0, r, 0)),
            pl.BlockSpec((2, 128), lambda r: (0, 0)),
        ],
        out_specs=[
            pl.BlockSpec((2, _RB, 128), lambda r: (0, r, 0)),
            pl.BlockSpec((1, 2, 128), lambda r: (r, 0, 0)),
            pl.BlockSpec((1, 2, 128), lambda r: (r, 0, 0)),
        ],
        out_shape=[
            jax.ShapeDtypeStruct((2, _N, 128), f32),
            jax.ShapeDtypeStruct((_NBLK, 2, 128), f32),
            jax.ShapeDtypeStruct((_NBLK, 2, 128), f32),
        ],
    )

    def k3_body(z_ref, sum_ref, sq_ref, gam_ref, bet_ref, w2_ref, deg_ref,
                out_ref):
        inv_n = 1.0 / _N
        mu = jnp.sum(sum_ref[...], axis=0) * inv_n
        ex2 = jnp.sum(sq_ref[...], axis=0) * inv_n
        sinv = lax.rsqrt(ex2 - mu * mu + 1e-5)
        h0 = jnp.maximum(
            (z_ref[0] - mu[0]) * (sinv[0] * gam_ref[0]) + bet_ref[0], 0.0)
        h1 = jnp.maximum(
            (z_ref[1] - mu[1]) * (sinv[1] * gam_ref[1]) + bet_ref[1], 0.0)
        g = (jnp.dot(h0, w2_ref[0], preferred_element_type=f32)
             + jnp.dot(h1, w2_ref[1], preferred_element_type=f32))
        dsum = deg_ref[0, :, 0] + deg_ref[1, :, 0] + 1.0
        out_ref[...] = g * lax.rsqrt(dsum)[:, None]

    k3 = pl.pallas_call(
        k3_body,
        grid=(_NBLK,),
        in_specs=[
            pl.BlockSpec((2, _RB, 128), lambda r: (0, r, 0)),
            pl.BlockSpec((_NBLK, 2, 128), lambda r: (0, 0, 0)),
            pl.BlockSpec((_NBLK, 2, 128), lambda r: (0, 0, 0)),
            pl.BlockSpec((2, 128), lambda r: (0, 0)),
            pl.BlockSpec((2, 128), lambda r: (0, 0)),
            pl.BlockSpec((2, 128, 128), lambda r: (0, 0, 0)),
            pl.BlockSpec((2, _RB, 128), lambda r: (0, r, 0)),
        ],
        out_specs=pl.BlockSpec((_RB, 128), lambda r: (r, 0)),
        out_shape=jax.ShapeDtypeStruct((_N, 128), f32),
    )

    def k4_body(a2_ref, g_ref, deg_ref, b2_ref, out_ref):
        dsum = deg_ref[0, :, 0] + deg_ref[1, :, 0] + 1.0
        dinv = lax.rsqrt(dsum)[:, None]
        acc = a2_ref[0, :, :40] + a2_ref[1, :, :40] + g_ref[:, :40]
        out_ref[...] = acc * dinv + b2_ref[0]

    k4 = pl.pallas_call(
        k4_body,
        grid=(_NBLK,),
        in_specs=[
            pl.BlockSpec((2, _RB, 128), lambda r: (0, r, 0)),
            pl.BlockSpec((_RB, 128), lambda r: (r, 0)),
            pl.BlockSpec((2, _RB, 128), lambda r: (0, r, 0)),
            pl.BlockSpec((1, 40), lambda r: (0, 0)),
        ],
        out_specs=pl.BlockSpec((_RB, 40), lambda r: (r, 0)),
        out_shape=jax.ShapeDtypeStruct((_N, 40), f32),
    )

    return sc_deg, sc_agg1, sc_agg2, k1, k2, k3, k4


def kernel(x, edge_index, W1, b1, gamma1, beta1, W2, b2):
    sc_deg, sc_agg1, sc_agg2, k1, k2, k3, k4 = _build()
    f32 = jnp.float32
    src = edge_index[0]
    dst = edge_index[1]
    pad = _EP - _E
    srcp = jnp.concatenate([src, jnp.zeros((pad,), jnp.int32)])
    dstp = jnp.concatenate([dst, jnp.full((pad,), _N, jnp.int32)])
    dstR32 = dstp.reshape(32, 40, _B)
    srcR32 = srcp.reshape(32, 40, _B)
    src2R = jnp.stack([srcp, srcp + _N]).reshape(2, 16, 80, _B)
    dstR16 = dstp.reshape(16, 80, _B)
    ones128 = jnp.ones((_B, 128), f32)
    zero128 = jnp.zeros((_NP, 128), f32)
    w2p = jnp.concatenate(
        [W2, jnp.zeros((256, 128 - 40), f32)], axis=1).reshape(2, 128, 128)

    deg = sc_deg(dstR32, ones128, zero128)
    h2 = k1(x, W1, deg)
    a1 = sc_agg1(h2.reshape(2 * _N, 128), src2R, dstR16, zero128)
    z, sums, sumsq = k2(a1, h2, deg, b1.reshape(2, 128))
    gp = k3(z, sums, sumsq, gamma1.reshape(2, 128), beta1.reshape(2, 128),
            w2p, deg)
    a2 = sc_agg2(gp, srcR32, dstR32, zero128)
    out = k4(a2, gp, deg, b2.reshape(1, 40))
    return out
